# baseline (speedup 1.0000x reference)
import functools

import jax
import jax.numpy as jnp
from jax import lax
from jax.experimental import pallas as pl
from jax.experimental.pallas import tpu as pltpu
from jax.experimental.pallas import tpu_sc as plsc

D_MODEL = 1024
LANES = 16
CHUNK = 32


def _pe_add_kernel(n_rows: int):
    info = plsc.get_sparse_core_info()
    num_workers = info.num_cores * info.num_subcores
    rows_per_w = n_rows // num_workers
    n_chunks = rows_per_w // CHUNK
    mesh = plsc.VectorSubcoreMesh(core_axis_name="c", subcore_axis_name="s")

    @functools.partial(
        pl.kernel,
        mesh=mesh,
        out_type=jax.ShapeDtypeStruct((n_rows, D_MODEL), jnp.float32),
        scratch_types=[
            pltpu.VMEM((rows_per_w,), jnp.int32),
            pltpu.VMEM((CHUNK, D_MODEL), jnp.float32),
            pltpu.VMEM((CHUNK, D_MODEL), jnp.float32),
            pltpu.SemaphoreType.DMA,
        ],
    )
    def k(x_hbm, idx_hbm, pe_hbm, out_hbm, idx_v, xbuf, rows, sem):
        wid = lax.axis_index("s") * info.num_cores + lax.axis_index("c")
        base = wid * rows_per_w
        pltpu.sync_copy(idx_hbm.at[pl.ds(base, rows_per_w)], idx_v)

        def chunk_body(g, carry):
            off = base + g * CHUNK
            gcpy = pltpu.async_copy(
                pe_hbm.at[idx_v.at[pl.ds(g * CHUNK, CHUNK)]], rows, sem)
            pltpu.sync_copy(x_hbm.at[pl.ds(off, CHUNK)], xbuf)
            gcpy.wait()

            def row_body(r, c):
                for j in range(D_MODEL // LANES):
                    sl = pl.ds(j * LANES, LANES)
                    plsc.addupdate(xbuf.at[r, sl], rows[r, sl])
                return c

            lax.fori_loop(0, CHUNK, row_body, 0)
            pltpu.sync_copy(xbuf, out_hbm.at[pl.ds(off, CHUNK)])
            return carry

        lax.fori_loop(0, n_chunks, chunk_body, 0)

    return k


def kernel(input_embeddings, position_ids, pe):
    b, s, d = input_embeddings.shape
    n = b * s
    x2d = input_embeddings.reshape(n, d)
    idx = position_ids.reshape(n).astype(jnp.int32)
    out = _pe_add_kernel(n)(x2d, idx, pe)
    return out.reshape(b, s, d)


# speedup vs baseline: 1.5178x; 1.5178x over previous
import functools

import jax
import jax.numpy as jnp
from jax import lax
from jax.experimental import pallas as pl
from jax.experimental.pallas import tpu as pltpu
from jax.experimental.pallas import tpu_sc as plsc

D_MODEL = 1024
LANES = 16
CHUNK = 16


def _pe_add_kernel(n_rows: int):
    info = plsc.get_sparse_core_info()
    num_workers = info.num_cores * info.num_subcores
    rows_per_w = n_rows // num_workers
    n_chunks = rows_per_w // CHUNK
    mesh = plsc.VectorSubcoreMesh(core_axis_name="c", subcore_axis_name="s")

    @functools.partial(
        pl.kernel,
        mesh=mesh,
        out_type=jax.ShapeDtypeStruct((n_rows, D_MODEL), jnp.float32),
        scratch_types=[
            pltpu.VMEM((rows_per_w,), jnp.int32),
            pltpu.VMEM((CHUNK, D_MODEL), jnp.float32),
            pltpu.VMEM((CHUNK, D_MODEL), jnp.float32),
            pltpu.VMEM((CHUNK, D_MODEL), jnp.float32),
            pltpu.VMEM((CHUNK, D_MODEL), jnp.float32),
            pltpu.SemaphoreType.DMA,
            pltpu.SemaphoreType.DMA,
            pltpu.SemaphoreType.DMA,
            pltpu.SemaphoreType.DMA,
        ],
    )
    def k(x_hbm, idx_hbm, pe_hbm, out_hbm,
          idx_v, xbuf0, xbuf1, rows0, rows1, gsem0, gsem1, xsem0, xsem1):
        wid = lax.axis_index("s") * info.num_cores + lax.axis_index("c")
        base = wid * rows_per_w
        xbufs, rows_bufs = (xbuf0, xbuf1), (rows0, rows1)
        gsems, xsems = (gsem0, gsem1), (xsem0, xsem1)

        pltpu.sync_copy(idx_hbm.at[pl.ds(base, rows_per_w)], idx_v)

        def start_loads(c, b):
            off = base + c * CHUNK
            pltpu.make_async_copy(
                pe_hbm.at[idx_v.at[pl.ds(c * CHUNK, CHUNK)]],
                rows_bufs[b], gsems[b]).start()
            pltpu.make_async_copy(
                x_hbm.at[pl.ds(off, CHUNK)], xbufs[b], xsems[b]).start()

        def wait_loads(c, b):
            pltpu.make_async_copy(
                pe_hbm.at[idx_v.at[pl.ds(c * CHUNK, CHUNK)]],
                rows_bufs[b], gsems[b]).wait()
            pltpu.make_async_copy(
                x_hbm.at[pl.ds(base, CHUNK)], xbufs[b], xsems[b]).wait()

        start_loads(0, 0)

        def outer(i, carry):
            g = i * 2
            for b in range(2):
                cur = g + b
                wait_loads(cur, b)

                @pl.when(cur + 1 < n_chunks)
                def _():
                    start_loads(cur + 1, 1 - b)

                xb, rb = xbufs[b], rows_bufs[b]

                def row_body(r, c):
                    for j in range(D_MODEL // LANES):
                        sl = pl.ds(j * LANES, LANES)
                        plsc.addupdate(xb.at[r, sl], rb[r, sl])
                    return c

                lax.fori_loop(0, CHUNK, row_body, 0)
                pltpu.sync_copy(
                    xb, out_hbm.at[pl.ds(base + cur * CHUNK, CHUNK)])
            return carry

        lax.fori_loop(0, n_chunks // 2, outer, 0)

    return k


def kernel(input_embeddings, position_ids, pe):
    b, s, d = input_embeddings.shape
    n = b * s
    x2d = input_embeddings.reshape(n, d)
    idx = position_ids.reshape(n).astype(jnp.int32)
    out = _pe_add_kernel(n)(x2d, idx, pe)
    return out.reshape(b, s, d)
